# Initial kernel scaffold; baseline (speedup 1.0000x reference)
import jax
import jax.numpy as jnp
from jax.experimental import pallas as pl


def kernel(x, W, b, ctx_mod, context):
    raise NotImplementedError("write your pallas kernel here")


# speedup vs baseline: 1.6818x; 1.6818x over previous
import functools

import jax
import jax.numpy as jnp
from jax import lax
from jax.experimental import pallas as pl
from jax.experimental.pallas import tpu as pltpu
from jax.experimental.pallas import tpu_sc as plsc

_NR = 4
_B = 2048
_D = 128
_NCTX = 4096
_T = _NR * _B
_AVG_N = 50000.0

_TOK_BLK = 512
_N_TOK_BLKS = _T // _TOK_BLK


def _k1_body(x_ref, ctx_ref, cm_ref, argm_ref, mseg_ref, cn_ref):
    pid = pl.program_id(0)

    @pl.when(pid == 0)
    def _prep():
        ctx = ctx_ref[...]
        nrm = jnp.sqrt(jnp.sum(ctx * ctx, axis=1, keepdims=True))
        cn_ref[...] = ctx / (nrm + 1e-8)
        seg = lax.dot_general(ctx, cm_ref[...], (((1,), (1,)), ((), ())),
                              preferred_element_type=jnp.float32)
        mseg_ref[...] = jnp.max(seg, axis=1, keepdims=True)

    xs = x_ref[0]
    sim = lax.dot_general(xs, cn_ref[...], (((1,), (1,)), ((), ())),
                          preferred_element_type=jnp.float32)
    argm_ref[0] = jnp.argmax(sim, axis=1).astype(jnp.int32)[:, None]


def _k3_body(x_ref, w_ref, b_ref, a_ref, o_ref):
    w = w_ref[...]
    acc = None
    for n in range(_NR):
        h = jnp.dot(x_ref[n], w, preferred_element_type=jnp.float32) + b_ref[...]
        g = jax.nn.gelu(h) * a_ref[:, n:n + 1]
        acc = g if acc is None else acc + g
    o_ref[...] = acc * (1.0 / _NR)


def _make_k2(nc, ns):
    nw = nc * ns
    rows_w = _NCTX // nw
    tok_w = _T // nw
    chunk = 128
    n_chunks = tok_w // chunk
    mesh = plsc.VectorSubcoreMesh(core_axis_name="c", subcore_axis_name="s")

    @functools.partial(
        pl.kernel,
        out_type=(
            jax.ShapeDtypeStruct((_NCTX, _D), jnp.float32),
            jax.ShapeDtypeStruct((_T,), jnp.float32),
        ),
        mesh=mesh,
        compiler_params=pltpu.CompilerParams(needs_layout_passes=False),
        scratch_types=[
            pltpu.VMEM((chunk, _D), jnp.float32),
            pltpu.VMEM((chunk, _D), jnp.float32),
            pltpu.VMEM((chunk,), jnp.int32),
            pltpu.VMEM((_NCTX,), jnp.float32),
            pltpu.VMEM((chunk,), jnp.float32),
            pltpu.SemaphoreType.DMA,
        ],
    )
    def k2(ctx_hbm, x_hbm, argm_hbm, mseg_hbm, newctx_hbm, act_hbm,
           buf_v, x_v, idx_v, mseg_v, act_v, sem):
        wid = lax.axis_index("s") * nc + lax.axis_index("c")
        r0 = wid * rows_w
        for r in range(0, rows_w, chunk):
            pltpu.sync_copy(ctx_hbm.at[pl.ds(r0 + r, chunk)], buf_v)
            pltpu.sync_copy(buf_v, newctx_hbm.at[pl.ds(r0 + r, chunk)])
        pltpu.sync_copy(mseg_hbm, mseg_v)
        plsc.subcore_barrier()

        for c in range(n_chunks):
            base = wid * tok_w + c * chunk
            pltpu.sync_copy(argm_hbm.at[pl.ds(base, chunk)], idx_v)
            pltpu.async_copy(ctx_hbm.at[idx_v], buf_v, sem).wait()
            pltpu.sync_copy(x_hbm.at[pl.ds(base, chunk)], x_v)

            def row_body(i, carry):
                for j in range(_D // 16):
                    sl = (i, pl.ds(j * 16, 16))
                    buf_v[sl] = (buf_v[sl] * (_AVG_N - 1.0) + x_v[sl]) * (1.0 / _AVG_N)
                return carry

            lax.fori_loop(0, chunk, row_body, 0)

            for j in range(chunk // 16):
                idx16 = idx_v[pl.ds(j * 16, 16)]
                m = plsc.load_gather(mseg_v, [idx16])
                act_v[pl.ds(j * 16, 16)] = 1.0 / (1.0 + jnp.exp(-m))

            pltpu.async_copy(buf_v, newctx_hbm.at[idx_v], sem).wait()
            pltpu.sync_copy(act_v, act_hbm.at[pl.ds(base, chunk)])

    return k2


def kernel(x, W, b, ctx_mod, context):
    xf = jnp.reshape(x, (_T, _D))

    argm3, mseg2 = pl.pallas_call(
        _k1_body,
        grid=(_N_TOK_BLKS,),
        in_specs=[
            pl.BlockSpec((1, _TOK_BLK, _D), lambda i: (i, 0, 0)),
            pl.BlockSpec((_NCTX, _D), lambda i: (0, 0)),
            pl.BlockSpec((_NR, _D), lambda i: (0, 0)),
        ],
        out_specs=[
            pl.BlockSpec((1, _TOK_BLK, 1), lambda i: (i, 0, 0)),
            pl.BlockSpec((_NCTX, 1), lambda i: (0, 0)),
        ],
        out_shape=[
            jax.ShapeDtypeStruct((_N_TOK_BLKS, _TOK_BLK, 1), jnp.int32),
            jax.ShapeDtypeStruct((_NCTX, 1), jnp.float32),
        ],
        scratch_shapes=[pltpu.VMEM((_NCTX, _D), jnp.float32)],
    )(jnp.reshape(xf, (_N_TOK_BLKS, _TOK_BLK, _D)), context, ctx_mod)

    argm = jnp.reshape(argm3, (_T,))
    mseg = jnp.reshape(mseg2, (_NCTX,))

    info = plsc.get_sparse_core_info()
    k2 = _make_k2(info.num_cores, info.num_subcores)
    new_context, act = k2(context, xf, argm, mseg)

    act2 = jnp.transpose(jnp.reshape(act, (_NR, _B)))
    blk = 256
    x_out = pl.pallas_call(
        _k3_body,
        grid=(_B // blk,),
        in_specs=[
            pl.BlockSpec((_NR, blk, _D), lambda i: (0, i, 0)),
            pl.BlockSpec((_D, _D), lambda i: (0, 0)),
            pl.BlockSpec((1, _D), lambda i: (0, 0)),
            pl.BlockSpec((blk, _NR), lambda i: (i, 0)),
        ],
        out_specs=pl.BlockSpec((blk, _D), lambda i: (i, 0)),
        out_shape=jax.ShapeDtypeStruct((_B, _D), jnp.float32),
    )(x, W, jnp.reshape(b, (1, _D)), act2)

    return (x_out, new_context)
